# baseline (speedup 1.0000x reference)
import jax
import jax.numpy as jnp
from jax import lax
from jax.experimental import pallas as pl
from jax.experimental.pallas import tpu as pltpu
from jax.experimental.pallas import tpu_sc as plsc

N = 10000
E = 320000
D = 128
NC = 2
NS = 16
NW = NC * NS
EPW = E // NW
K = 128
CHT = E // K
CH = CHT // NW
XTRA = CHT - CH * NW
RPT = N // NS
Z8 = 624

f32 = jnp.float32

_mesh = plsc.VectorSubcoreMesh(
    core_axis_name="c", subcore_axis_name="s", num_cores=NC, num_subcores=NS)


def _deg_body(dst_hbm, degp_hbm, dstall, dxtra, ones_v, zb, dsem, degacc):
    cix = lax.axis_index("c")
    sid = lax.axis_index("s")
    w = cix * NS + sid
    z0 = sid * Z8
    tail0, tailn = NS * Z8, N - NS * Z8
    for i in range(Z8 // 16):
        zb[pl.ds(i * 16, 16)] = jnp.zeros((16,), f32)
    pltpu.sync_copy(zb, degacc.at[pl.ds(z0, Z8)])

    @pl.when(sid == NS - 1)
    def _():
        pltpu.sync_copy(zb.at[pl.ds(0, tailn)], degacc.at[pl.ds(tail0, tailn)])

    for i in range(K // 16):
        ones_v[pl.ds(i * 16, 16)] = jnp.ones((16,), f32)
    pltpu.sync_copy(dst_hbm.at[pl.ds(w * CH, CH)], dstall)

    @pl.when(w < XTRA)
    def _():
        pltpu.sync_copy(dst_hbm.at[pl.ds(NW * CH + w, 1)], dxtra)

    plsc.subcore_barrier()
    WAVE = 8
    for g0 in range(0, CH, WAVE):
        hs = [pltpu.async_copy(ones_v, degacc.at[dstall.at[ci]], dsem,
                               add=True)
              for ci in range(g0, min(g0 + WAVE, CH))]
        for h in hs:
            h.wait()

    @pl.when(w < XTRA)
    def _():
        pltpu.sync_copy(ones_v, degacc.at[dxtra.at[0]], add=True)

    plsc.subcore_barrier()
    pltpu.sync_copy(degacc.at[pl.ds(z0, Z8)], degp_hbm.at[cix, pl.ds(z0, Z8)])

    @pl.when(sid == NS - 1)
    def _():
        pltpu.sync_copy(degacc.at[pl.ds(tail0, tailn)],
                        degp_hbm.at[cix, pl.ds(tail0, tailn)])


_sc_params = pltpu.CompilerParams(use_tc_tiling_on_sc=False)

_deg_call = pl.kernel(
    _deg_body,
    out_type=jax.ShapeDtypeStruct((NC, N), f32),
    mesh=_mesh,
    compiler_params=_sc_params,
    scratch_types=[
        pltpu.VMEM((CH, K), jnp.int32),
        pltpu.VMEM((1, K), jnp.int32),
        pltpu.VMEM((K,), f32),
        pltpu.VMEM((Z8,), f32),
        pltpu.SemaphoreType.DMA,
        pltpu.VMEM_SHARED((N,), f32),
    ],
)


ZB = 25


def _spmm_body(xs_hbm, src_hbm, dst_hbm, accp_hbm,
               sidx0, sidx1, didx0, didx1, rows0, rows1, zb,
               isem0, isem1, gsem0, gsem1, acc):
    cix = lax.axis_index("c")
    sid = lax.axis_index("s")
    w = cix * NS + sid
    r0 = sid * RPT
    for j in range(ZB):
        for i in range(D // 16):
            zb[j, pl.ds(i * 16, 16)] = jnp.zeros((16,), f32)
    zhs = [pltpu.async_copy(zb, acc.at[pl.ds(r0 + j * ZB, ZB)], gsem0)
           for j in range(RPT // ZB)]

    sidx = (sidx0, sidx1)
    didx = (didx0, didx1)
    rows = (rows0, rows1)
    isems = (isem0, isem1)
    gsems = (gsem0, gsem1)

    def fire_idx(ci, k):
        row0 = w * CH + ci
        return [pltpu.async_copy(src_hbm.at[pl.ds(row0, 1)], sidx[k],
                                 isems[k]),
                pltpu.async_copy(dst_hbm.at[pl.ds(row0, 1)], didx[k],
                                 isems[k])]

    def fire_gather(k, ih):
        for h in ih:
            h.wait()
        return pltpu.async_copy(xs_hbm.at[sidx[k].at[0]], rows[k], gsems[k])

    def drain(k, h):
        h.wait()
        pltpu.sync_copy(rows[k], acc.at[didx[k].at[0]], add=True)

    ih = {0: fire_idx(0, 0)}
    for zh in zhs:
        zh.wait()
    gh = {0: fire_gather(0, ih[0])}
    ih[1] = fire_idx(1, 1)
    plsc.subcore_barrier()
    for ci in range(1, CH):
        k = ci % 2
        p = 1 - k
        gh[k] = fire_gather(k, ih[k])
        drain(p, gh[p])
        if ci + 1 < CH:
            ih[p] = fire_idx(ci + 1, p)
    drain((CH - 1) % 2, gh[(CH - 1) % 2])

    @pl.when(w < XTRA)
    def _():
        row0 = NW * CH + w
        pltpu.sync_copy(src_hbm.at[pl.ds(row0, 1)], sidx[0])
        pltpu.sync_copy(dst_hbm.at[pl.ds(row0, 1)], didx[0])
        pltpu.async_copy(xs_hbm.at[sidx[0].at[0]], rows[0], gsems[0]).wait()
        pltpu.sync_copy(rows[0], acc.at[didx[0].at[0]], add=True)

    plsc.subcore_barrier()
    pltpu.sync_copy(acc.at[pl.ds(r0, RPT)], accp_hbm.at[cix, pl.ds(r0, RPT)])


_spmm_call = pl.kernel(
    _spmm_body,
    out_type=jax.ShapeDtypeStruct((NC, N, D), f32),
    mesh=_mesh,
    compiler_params=_sc_params,
    scratch_types=[
        pltpu.VMEM((1, K), jnp.int32),
        pltpu.VMEM((1, K), jnp.int32),
        pltpu.VMEM((1, K), jnp.int32),
        pltpu.VMEM((1, K), jnp.int32),
        pltpu.VMEM((K, D), f32),
        pltpu.VMEM((K, D), f32),
        pltpu.VMEM((ZB, D), f32),
        pltpu.SemaphoreType.DMA,
        pltpu.SemaphoreType.DMA,
        pltpu.SemaphoreType.DMA,
        pltpu.SemaphoreType.DMA,
        pltpu.VMEM_SHARED((N, D), f32),
    ],
)

R = 2000
G = N // R


def _mm1_body(rep, w1, xw1):
    xw1[...] = jnp.dot(rep[...], w1[...], preferred_element_type=f32)


def _mm1(rep, W1):
    return pl.pallas_call(
        _mm1_body,
        grid=(G,),
        in_specs=[
            pl.BlockSpec((R, D), lambda i: (i, 0)),
            pl.BlockSpec((D, D), lambda i: (0, 0)),
        ],
        out_specs=pl.BlockSpec((R, D), lambda i: (i, 0)),
        out_shape=jax.ShapeDtypeStruct((N, D), f32),
    )(rep, W1)


def _tpart_body(ind, g, w1a_t, w1g_t, b1_t, w1a_c, w1g_c, b1_c, tpt, tpc):
    iv = ind[...]
    gg = g[...]
    tpt[...] = (jnp.dot(iv, w1a_t[...], preferred_element_type=f32)
                + gg * w1g_t[...] + b1_t[...])
    tpc[...] = (jnp.dot(iv, w1a_c[...], preferred_element_type=f32)
                + gg * w1g_c[...] + b1_c[...])


def _tpart(ind_rep, g, w1a_t, w1g_t, b1_t, w1a_c, w1g_c, b1_c):
    return pl.pallas_call(
        _tpart_body,
        grid=(G,),
        in_specs=[
            pl.BlockSpec((R, D), lambda i: (i, 0)),
            pl.BlockSpec((R, 1), lambda i: (i, 0)),
            pl.BlockSpec((D, D), lambda i: (0, 0)),
            pl.BlockSpec((1, D), lambda i: (0, 0)),
            pl.BlockSpec((1, D), lambda i: (0, 0)),
            pl.BlockSpec((D, D), lambda i: (0, 0)),
            pl.BlockSpec((1, D), lambda i: (0, 0)),
            pl.BlockSpec((1, D), lambda i: (0, 0)),
        ],
        out_specs=[
            pl.BlockSpec((R, D), lambda i: (i, 0)),
            pl.BlockSpec((R, D), lambda i: (i, 0)),
        ],
        out_shape=[
            jax.ShapeDtypeStruct((N, D), f32),
            jax.ShapeDtypeStruct((N, D), f32),
        ],
    )(ind_rep, g, w1a_t, w1g_t, b1_t, w1a_c, w1g_c, b1_c)


def _scale1_body(xw1, dega, degb, xs1, dis):
    d = lax.rsqrt(dega[...] + degb[...] + 2.0)
    dis[...] = d
    xs1[...] = d * xw1[...]


def _scale1(xw1, dega, degb):
    return pl.pallas_call(
        _scale1_body,
        grid=(G,),
        in_specs=[
            pl.BlockSpec((R, D), lambda i: (i, 0)),
            pl.BlockSpec((R, 1), lambda i: (i, 0)),
            pl.BlockSpec((R, 1), lambda i: (i, 0)),
        ],
        out_specs=[
            pl.BlockSpec((R, D), lambda i: (i, 0)),
            pl.BlockSpec((R, 1), lambda i: (i, 0)),
        ],
        out_shape=[
            jax.ShapeDtypeStruct((N, D), f32),
            jax.ShapeDtypeStruct((N, 1), f32),
        ],
    )(xw1, dega, degb)


def _prep2_body(acca, accb, xs1, dis, b1, w2, xs2):
    d = dis[...]
    h1 = jnp.maximum(
        d * (acca[...][0] + accb[...][0] + 2.0 * xs1[...]) + b1[...], 0.0)
    xs2[...] = d * jnp.dot(h1, w2[...], preferred_element_type=f32)


def _prep2(accp, xs1, dis, b1, W2):
    return pl.pallas_call(
        _prep2_body,
        grid=(G,),
        in_specs=[
            pl.BlockSpec((1, R, D), lambda i: (0, i, 0)),
            pl.BlockSpec((1, R, D), lambda i: (1, i, 0)),
            pl.BlockSpec((R, D), lambda i: (i, 0)),
            pl.BlockSpec((R, 1), lambda i: (i, 0)),
            pl.BlockSpec((1, D), lambda i: (0, 0)),
            pl.BlockSpec((D, D), lambda i: (0, 0)),
        ],
        out_specs=pl.BlockSpec((R, D), lambda i: (i, 0)),
        out_shape=jax.ShapeDtypeStruct((N, D), f32),
    )(accp, accp, xs1, dis, b1, W2)


def _final_body(tpt, tpc, acca, accb, xs2, dis, b2, tcm,
                wt1b, wt2, bt2, wt3, bt3,
                wc1b, wc2, bc2, wc3, bc3,
                spill, h2o):
    d = dis[...]
    h2 = jnp.maximum(
        d * (acca[...][0] + accb[...][0] + 2.0 * xs2[...]) + b2[...], 0.0)
    h2o[...] = h2

    def head(tp, w1b, w2_, bb2, w3r, bb3):
        t1 = jnp.maximum(
            tp[...] + jnp.dot(h2, w1b[...], preferred_element_type=f32), 0.0)
        t2 = jnp.maximum(
            jnp.dot(t1, w2_[...], preferred_element_type=f32) + bb2[...], 0.0)
        return jnp.sum(t2 * w3r[...], axis=1, keepdims=True) + bb3[...]

    st = head(tpt, wt1b, wt2, bt2, wt3, bt3)
    sc = head(tpc, wc1b, wc2, bc2, wc3, bc3)
    spill[...] = jnp.where(tcm[...] == 1, st, sc)


def _final(tpt, tpc, accp, xs2, dis, b2, tcm, wts):
    H2M, H3 = 128, 64
    row = lambda: pl.BlockSpec((R, D), lambda i: (i, 0))
    col = lambda: pl.BlockSpec((R, 1), lambda i: (i, 0))
    full = lambda a, b: pl.BlockSpec((a, b), lambda i: (0, 0))
    return pl.pallas_call(
        _final_body,
        grid=(G,),
        in_specs=[
            row(), row(),
            pl.BlockSpec((1, R, D), lambda i: (0, i, 0)),
            pl.BlockSpec((1, R, D), lambda i: (1, i, 0)),
            row(), col(), full(1, D), col(),
            full(D, H2M), full(H2M, H3), full(1, H3), full(1, H3), full(1, 1),
            full(D, H2M), full(H2M, H3), full(1, H3), full(1, H3), full(1, 1),
        ],
        out_specs=[
            pl.BlockSpec((R, 1), lambda i: (i, 0)),
            pl.BlockSpec((R, D), lambda i: (i, 0)),
        ],
        out_shape=[
            jax.ShapeDtypeStruct((N, 1), f32),
            jax.ShapeDtypeStruct((N, D), f32),
        ],
    )(tpt, tpc, accp, accp, xs2, dis, b2, tcm, *wts)


def kernel(ind_rep, rep, tc, g, edge_index, W1, b1, W2, b2,
           Wt1, bt1, Wt2, bt2, Wt3, bt3, Wc1, bc1, Wc2, bc2, Wc3, bc3):
    ei = edge_index.astype(jnp.int32)
    src2d = ei[0].reshape(CHT, K)
    dst2d = ei[1].reshape(CHT, K)

    degp = _deg_call(dst2d)
    dega = degp[0].reshape(N, 1)
    degb = degp[1].reshape(N, 1)

    xw1 = _mm1(rep, W1)
    tpt, tpc = _tpart(ind_rep, g.reshape(N, 1),
                      Wt1[:D], Wt1[2 * D:].reshape(1, D), bt1.reshape(1, D),
                      Wc1[:D], Wc1[2 * D:].reshape(1, D), bc1.reshape(1, D))
    xs1, dis = _scale1(xw1, dega, degb)
    accp1 = _spmm_call(xs1, src2d, dst2d)
    xs2 = _prep2(accp1, xs1, dis, b1.reshape(1, D), W2)
    accp2 = _spmm_call(xs2, src2d, dst2d)

    wts = (Wt1[D:2 * D], Wt2, bt2.reshape(1, 64),
           Wt3.reshape(1, 64), bt3.reshape(1, 1),
           Wc1[D:2 * D], Wc2, bc2.reshape(1, 64),
           Wc3.reshape(1, 64), bc3.reshape(1, 1))
    spill, h2 = _final(tpt, tpc, accp2, xs2, dis,
                       b2.reshape(1, D),
                       tc.astype(jnp.int32).reshape(N, 1), wts)
    return jnp.squeeze(spill, -1), h2


# speedup vs baseline: 1.0491x; 1.0491x over previous
import jax
import jax.numpy as jnp
from jax import lax
from jax.experimental import pallas as pl
from jax.experimental.pallas import tpu as pltpu
from jax.experimental.pallas import tpu_sc as plsc

N = 10000
E = 320000
D = 128
NC = 2
NS = 16
NW = NC * NS
EPW = E // NW
K = 128
CHT = E // K
CH = CHT // NW
XTRA = CHT - CH * NW
KS = 80
CHS = EPW // KS
SUP = 2
NSUP = (CHS + SUP - 1) // SUP
RPT = N // NS
Z8 = 624

f32 = jnp.float32

_mesh = plsc.VectorSubcoreMesh(
    core_axis_name="c", subcore_axis_name="s", num_cores=NC, num_subcores=NS)


def _deg_body(dst_hbm, degp_hbm, dstall, dxtra, ones_v, zb, dsem, degacc):
    cix = lax.axis_index("c")
    sid = lax.axis_index("s")
    w = cix * NS + sid
    z0 = sid * Z8
    tail0, tailn = NS * Z8, N - NS * Z8
    for i in range(Z8 // 16):
        zb[pl.ds(i * 16, 16)] = jnp.zeros((16,), f32)
    pltpu.sync_copy(zb, degacc.at[pl.ds(z0, Z8)])

    @pl.when(sid == NS - 1)
    def _():
        pltpu.sync_copy(zb.at[pl.ds(0, tailn)], degacc.at[pl.ds(tail0, tailn)])

    for i in range(K // 16):
        ones_v[pl.ds(i * 16, 16)] = jnp.ones((16,), f32)
    pltpu.sync_copy(dst_hbm.at[pl.ds(w * CH, CH)], dstall)

    @pl.when(w < XTRA)
    def _():
        pltpu.sync_copy(dst_hbm.at[pl.ds(NW * CH + w, 1)], dxtra)

    plsc.subcore_barrier()
    WAVE = 8
    for g0 in range(0, CH, WAVE):
        hs = [pltpu.async_copy(ones_v, degacc.at[dstall.at[ci]], dsem,
                               add=True)
              for ci in range(g0, min(g0 + WAVE, CH))]
        for h in hs:
            h.wait()

    @pl.when(w < XTRA)
    def _():
        pltpu.sync_copy(ones_v, degacc.at[dxtra.at[0]], add=True)

    plsc.subcore_barrier()
    pltpu.sync_copy(degacc.at[pl.ds(z0, Z8)], degp_hbm.at[cix, pl.ds(z0, Z8)])

    @pl.when(sid == NS - 1)
    def _():
        pltpu.sync_copy(degacc.at[pl.ds(tail0, tailn)],
                        degp_hbm.at[cix, pl.ds(tail0, tailn)])


_sc_params = pltpu.CompilerParams(use_tc_tiling_on_sc=False)

_deg_call = pl.kernel(
    _deg_body,
    out_type=jax.ShapeDtypeStruct((NC, N), f32),
    mesh=_mesh,
    compiler_params=_sc_params,
    scratch_types=[
        pltpu.VMEM((CH, K), jnp.int32),
        pltpu.VMEM((1, K), jnp.int32),
        pltpu.VMEM((K,), f32),
        pltpu.VMEM((Z8,), f32),
        pltpu.SemaphoreType.DMA,
        pltpu.VMEM_SHARED((N,), f32),
    ],
)


ZB = 25


def _spmm_body(xs_hbm, src_hbm, dst_hbm, accp_hbm,
               sidx0, sidx1, didx0, didx1, rows0, rows1, zb,
               isem0, isem1, gsem0, gsem1, acc):
    cix = lax.axis_index("c")
    sid = lax.axis_index("s")
    w = cix * NS + sid
    r0 = sid * RPT
    for j in range(ZB):
        for i in range(D // 16):
            zb[j, pl.ds(i * 16, 16)] = jnp.zeros((16,), f32)
    zhs = [pltpu.async_copy(zb, acc.at[pl.ds(r0 + j * ZB, ZB)], gsem0)
           for j in range(RPT // ZB)]

    sidx = (sidx0, sidx1)
    didx = (didx0, didx1)
    rows = (rows0, rows1)
    isems = (isem0, isem1)
    gsems = (gsem0, gsem1)

    def nch(si):
        return min(SUP, CHS - si * SUP)

    def fire_idx(si, k):
        row0 = w * CHS + si * SUP
        nc = nch(si)
        return [pltpu.async_copy(src_hbm.at[pl.ds(row0, nc)],
                                 sidx[k].at[pl.ds(0, nc)], isems[k]),
                pltpu.async_copy(dst_hbm.at[pl.ds(row0, nc)],
                                 didx[k].at[pl.ds(0, nc)], isems[k])]

    def fire_gather(si, k, ih):
        for h in ih:
            h.wait()
        return [pltpu.async_copy(xs_hbm.at[sidx[k].at[c]],
                                 rows[k].at[c], gsems[k])
                for c in range(nch(si))]

    def drain(si, k, hs):
        for h in hs:
            h.wait()
        for c in range(nch(si)):
            pltpu.sync_copy(rows[k].at[c], acc.at[didx[k].at[c]], add=True)

    ih = fire_idx(0, 0)
    for zh in zhs:
        zh.wait()
    gh = {0: fire_gather(0, 0, ih)}
    ih = fire_idx(1, 1)
    plsc.subcore_barrier()
    for si in range(1, NSUP):
        k = si % 2
        p = 1 - k
        gh[k] = fire_gather(si, k, ih)
        drain(si - 1, p, gh[p])
        if si + 1 < NSUP:
            ih = fire_idx(si + 1, p)
    drain(NSUP - 1, (NSUP - 1) % 2, gh[(NSUP - 1) % 2])

    plsc.subcore_barrier()
    pltpu.sync_copy(acc.at[pl.ds(r0, RPT)], accp_hbm.at[cix, pl.ds(r0, RPT)])


_spmm_call = pl.kernel(
    _spmm_body,
    out_type=jax.ShapeDtypeStruct((NC, N, D), f32),
    mesh=_mesh,
    compiler_params=_sc_params,
    scratch_types=[
        pltpu.VMEM((SUP, KS), jnp.int32),
        pltpu.VMEM((SUP, KS), jnp.int32),
        pltpu.VMEM((SUP, KS), jnp.int32),
        pltpu.VMEM((SUP, KS), jnp.int32),
        pltpu.VMEM((SUP, KS, D), f32),
        pltpu.VMEM((SUP, KS, D), f32),
        pltpu.VMEM((ZB, D), f32),
        pltpu.SemaphoreType.DMA,
        pltpu.SemaphoreType.DMA,
        pltpu.SemaphoreType.DMA,
        pltpu.SemaphoreType.DMA,
        pltpu.VMEM_SHARED((N, D), f32),
    ],
)

R = 2000
G = N // R


def _mm1_body(rep, w1, xw1):
    xw1[...] = jnp.dot(rep[...], w1[...], preferred_element_type=f32)


def _mm1(rep, W1):
    return pl.pallas_call(
        _mm1_body,
        grid=(G,),
        in_specs=[
            pl.BlockSpec((R, D), lambda i: (i, 0)),
            pl.BlockSpec((D, D), lambda i: (0, 0)),
        ],
        out_specs=pl.BlockSpec((R, D), lambda i: (i, 0)),
        out_shape=jax.ShapeDtypeStruct((N, D), f32),
    )(rep, W1)


def _tpart_body(ind, g, w1a_t, w1g_t, b1_t, w1a_c, w1g_c, b1_c, tpt, tpc):
    iv = ind[...]
    gg = g[...]
    tpt[...] = (jnp.dot(iv, w1a_t[...], preferred_element_type=f32)
                + gg * w1g_t[...] + b1_t[...])
    tpc[...] = (jnp.dot(iv, w1a_c[...], preferred_element_type=f32)
                + gg * w1g_c[...] + b1_c[...])


def _tpart(ind_rep, g, w1a_t, w1g_t, b1_t, w1a_c, w1g_c, b1_c):
    return pl.pallas_call(
        _tpart_body,
        grid=(G,),
        in_specs=[
            pl.BlockSpec((R, D), lambda i: (i, 0)),
            pl.BlockSpec((R, 1), lambda i: (i, 0)),
            pl.BlockSpec((D, D), lambda i: (0, 0)),
            pl.BlockSpec((1, D), lambda i: (0, 0)),
            pl.BlockSpec((1, D), lambda i: (0, 0)),
            pl.BlockSpec((D, D), lambda i: (0, 0)),
            pl.BlockSpec((1, D), lambda i: (0, 0)),
            pl.BlockSpec((1, D), lambda i: (0, 0)),
        ],
        out_specs=[
            pl.BlockSpec((R, D), lambda i: (i, 0)),
            pl.BlockSpec((R, D), lambda i: (i, 0)),
        ],
        out_shape=[
            jax.ShapeDtypeStruct((N, D), f32),
            jax.ShapeDtypeStruct((N, D), f32),
        ],
    )(ind_rep, g, w1a_t, w1g_t, b1_t, w1a_c, w1g_c, b1_c)


def _scale1_body(xw1, dega, degb, xs1, dis):
    d = lax.rsqrt(dega[...] + degb[...] + 2.0)
    dis[...] = d
    xs1[...] = d * xw1[...]


def _scale1(xw1, dega, degb):
    return pl.pallas_call(
        _scale1_body,
        grid=(G,),
        in_specs=[
            pl.BlockSpec((R, D), lambda i: (i, 0)),
            pl.BlockSpec((R, 1), lambda i: (i, 0)),
            pl.BlockSpec((R, 1), lambda i: (i, 0)),
        ],
        out_specs=[
            pl.BlockSpec((R, D), lambda i: (i, 0)),
            pl.BlockSpec((R, 1), lambda i: (i, 0)),
        ],
        out_shape=[
            jax.ShapeDtypeStruct((N, D), f32),
            jax.ShapeDtypeStruct((N, 1), f32),
        ],
    )(xw1, dega, degb)


def _prep2_body(acca, accb, xs1, dis, b1, w2, xs2):
    d = dis[...]
    h1 = jnp.maximum(
        d * (acca[...][0] + accb[...][0] + 2.0 * xs1[...]) + b1[...], 0.0)
    xs2[...] = d * jnp.dot(h1, w2[...], preferred_element_type=f32)


def _prep2(accp, xs1, dis, b1, W2):
    return pl.pallas_call(
        _prep2_body,
        grid=(G,),
        in_specs=[
            pl.BlockSpec((1, R, D), lambda i: (0, i, 0)),
            pl.BlockSpec((1, R, D), lambda i: (1, i, 0)),
            pl.BlockSpec((R, D), lambda i: (i, 0)),
            pl.BlockSpec((R, 1), lambda i: (i, 0)),
            pl.BlockSpec((1, D), lambda i: (0, 0)),
            pl.BlockSpec((D, D), lambda i: (0, 0)),
        ],
        out_specs=pl.BlockSpec((R, D), lambda i: (i, 0)),
        out_shape=jax.ShapeDtypeStruct((N, D), f32),
    )(accp, accp, xs1, dis, b1, W2)


def _final_body(tpt, tpc, acca, accb, xs2, dis, b2, tcm,
                wt1b, wt2, bt2, wt3, bt3,
                wc1b, wc2, bc2, wc3, bc3,
                spill, h2o):
    d = dis[...]
    h2 = jnp.maximum(
        d * (acca[...][0] + accb[...][0] + 2.0 * xs2[...]) + b2[...], 0.0)
    h2o[...] = h2

    def head(tp, w1b, w2_, bb2, w3r, bb3):
        t1 = jnp.maximum(
            tp[...] + jnp.dot(h2, w1b[...], preferred_element_type=f32), 0.0)
        t2 = jnp.maximum(
            jnp.dot(t1, w2_[...], preferred_element_type=f32) + bb2[...], 0.0)
        return jnp.sum(t2 * w3r[...], axis=1, keepdims=True) + bb3[...]

    st = head(tpt, wt1b, wt2, bt2, wt3, bt3)
    sc = head(tpc, wc1b, wc2, bc2, wc3, bc3)
    spill[...] = jnp.where(tcm[...] == 1, st, sc)


def _final(tpt, tpc, accp, xs2, dis, b2, tcm, wts):
    H2M, H3 = 128, 64
    row = lambda: pl.BlockSpec((R, D), lambda i: (i, 0))
    col = lambda: pl.BlockSpec((R, 1), lambda i: (i, 0))
    full = lambda a, b: pl.BlockSpec((a, b), lambda i: (0, 0))
    return pl.pallas_call(
        _final_body,
        grid=(G,),
        in_specs=[
            row(), row(),
            pl.BlockSpec((1, R, D), lambda i: (0, i, 0)),
            pl.BlockSpec((1, R, D), lambda i: (1, i, 0)),
            row(), col(), full(1, D), col(),
            full(D, H2M), full(H2M, H3), full(1, H3), full(1, H3), full(1, 1),
            full(D, H2M), full(H2M, H3), full(1, H3), full(1, H3), full(1, 1),
        ],
        out_specs=[
            pl.BlockSpec((R, 1), lambda i: (i, 0)),
            pl.BlockSpec((R, D), lambda i: (i, 0)),
        ],
        out_shape=[
            jax.ShapeDtypeStruct((N, 1), f32),
            jax.ShapeDtypeStruct((N, D), f32),
        ],
    )(tpt, tpc, accp, accp, xs2, dis, b2, tcm, *wts)


def kernel(ind_rep, rep, tc, g, edge_index, W1, b1, W2, b2,
           Wt1, bt1, Wt2, bt2, Wt3, bt3, Wc1, bc1, Wc2, bc2, Wc3, bc3):
    ei = edge_index.astype(jnp.int32)
    src2d = ei[0].reshape(NW * CHS, KS)
    dst2d = ei[1].reshape(NW * CHS, KS)
    dst2k = ei[1].reshape(CHT, K)

    degp = _deg_call(dst2k)
    dega = degp[0].reshape(N, 1)
    degb = degp[1].reshape(N, 1)

    xw1 = _mm1(rep, W1)
    tpt, tpc = _tpart(ind_rep, g.reshape(N, 1),
                      Wt1[:D], Wt1[2 * D:].reshape(1, D), bt1.reshape(1, D),
                      Wc1[:D], Wc1[2 * D:].reshape(1, D), bc1.reshape(1, D))
    xs1, dis = _scale1(xw1, dega, degb)
    accp1 = _spmm_call(xs1, src2d, dst2d)
    xs2 = _prep2(accp1, xs1, dis, b1.reshape(1, D), W2)
    accp2 = _spmm_call(xs2, src2d, dst2d)

    wts = (Wt1[D:2 * D], Wt2, bt2.reshape(1, 64),
           Wt3.reshape(1, 64), bt3.reshape(1, 1),
           Wc1[D:2 * D], Wc2, bc2.reshape(1, 64),
           Wc3.reshape(1, 64), bc3.reshape(1, 1))
    spill, h2 = _final(tpt, tpc, accp2, xs2, dis,
                       b2.reshape(1, D),
                       tc.astype(jnp.int32).reshape(N, 1), wts)
    return jnp.squeeze(spill, -1), h2


# speedup vs baseline: 1.1583x; 1.1042x over previous
import jax
import jax.numpy as jnp
from jax import lax
from jax.experimental import pallas as pl
from jax.experimental.pallas import tpu as pltpu
from jax.experimental.pallas import tpu_sc as plsc

N = 10000
E = 320000
D = 128
NC = 2
NS = 16
NW = NC * NS
EPW = E // NW
KS = 80
CHS = EPW // KS
SUP = 2
NSUP = (CHS + SUP - 1) // SUP
RPT = N // NS
Z8 = 624

f32 = jnp.float32

_mesh = plsc.VectorSubcoreMesh(
    core_axis_name="c", subcore_axis_name="s", num_cores=NC, num_subcores=NS)


def _deg_body(er_hbm, degp0_hbm, degp1_hbm, dstall, ones_v, zb, dsem, degacc):
    cix = lax.axis_index("c")
    sid = lax.axis_index("s")
    w = cix * NS + sid
    z0 = sid * Z8
    tail0, tailn = NS * Z8, N - NS * Z8
    for i in range(Z8 // 16):
        zb[pl.ds(i * 16, 16)] = jnp.zeros((16,), f32)
    pltpu.sync_copy(zb, degacc.at[pl.ds(z0, Z8)])

    @pl.when(sid == NS - 1)
    def _():
        pltpu.sync_copy(zb.at[pl.ds(0, tailn)], degacc.at[pl.ds(tail0, tailn)])

    for i in range(KS // 16):
        ones_v[pl.ds(i * 16, 16)] = jnp.ones((16,), f32)
    pltpu.sync_copy(er_hbm.at[1, pl.ds(w * CHS, CHS)], dstall)
    plsc.subcore_barrier()
    WAVE = 8
    for g0 in range(0, CHS, WAVE):
        hs = [pltpu.async_copy(ones_v, degacc.at[dstall.at[ci]], dsem,
                               add=True)
              for ci in range(g0, min(g0 + WAVE, CHS))]
        for h in hs:
            h.wait()

    plsc.subcore_barrier()
    degp = (degp0_hbm, degp1_hbm)
    for c in range(NC):
        @pl.when(cix == c)
        def _():
            pltpu.sync_copy(degacc.at[pl.ds(z0, Z8)],
                            degp[c].at[pl.ds(z0, Z8)])

            @pl.when(sid == NS - 1)
            def _():
                pltpu.sync_copy(degacc.at[pl.ds(tail0, tailn)],
                                degp[c].at[pl.ds(tail0, tailn)])


_sc_params = pltpu.CompilerParams(use_tc_tiling_on_sc=False)

_deg_call = pl.kernel(
    _deg_body,
    out_type=(jax.ShapeDtypeStruct((N,), f32), jax.ShapeDtypeStruct((N,), f32)),
    mesh=_mesh,
    compiler_params=_sc_params,
    scratch_types=[
        pltpu.VMEM((CHS, KS), jnp.int32),
        pltpu.VMEM((KS,), f32),
        pltpu.VMEM((Z8,), f32),
        pltpu.SemaphoreType.DMA,
        pltpu.VMEM_SHARED((N,), f32),
    ],
)


ZB = 25


def _spmm_body(xs_hbm, er_hbm, accp_hbm,
               sidx0, sidx1, didx0, didx1, rows0, rows1, zb,
               isem0, isem1, gsem0, gsem1, acc):
    cix = lax.axis_index("c")
    sid = lax.axis_index("s")
    w = cix * NS + sid
    r0 = sid * RPT
    for j in range(ZB):
        for i in range(D // 16):
            zb[j, pl.ds(i * 16, 16)] = jnp.zeros((16,), f32)
    zhs = [pltpu.async_copy(zb, acc.at[pl.ds(r0 + j * ZB, ZB)], gsem0)
           for j in range(RPT // ZB)]

    sidx = (sidx0, sidx1)
    didx = (didx0, didx1)
    rows = (rows0, rows1)
    isems = (isem0, isem1)
    gsems = (gsem0, gsem1)

    def nch(si):
        return min(SUP, CHS - si * SUP)

    def fire_idx(si, k):
        row0 = w * CHS + si * SUP
        nc = nch(si)
        return [pltpu.async_copy(er_hbm.at[0, pl.ds(row0, nc)],
                                 sidx[k].at[pl.ds(0, nc)], isems[k]),
                pltpu.async_copy(er_hbm.at[1, pl.ds(row0, nc)],
                                 didx[k].at[pl.ds(0, nc)], isems[k])]

    def fire_gather(si, k, ih):
        for h in ih:
            h.wait()
        return [pltpu.async_copy(xs_hbm.at[sidx[k].at[c]],
                                 rows[k].at[c], gsems[k])
                for c in range(nch(si))]

    def drain(si, k, hs):
        for h in hs:
            h.wait()
        for c in range(nch(si)):
            pltpu.sync_copy(rows[k].at[c], acc.at[didx[k].at[c]], add=True)

    ih = fire_idx(0, 0)
    for zh in zhs:
        zh.wait()
    gh = {0: fire_gather(0, 0, ih)}
    ih = fire_idx(1, 1)
    plsc.subcore_barrier()
    for si in range(1, NSUP):
        k = si % 2
        p = 1 - k
        gh[k] = fire_gather(si, k, ih)
        drain(si - 1, p, gh[p])
        if si + 1 < NSUP:
            ih = fire_idx(si + 1, p)
    drain(NSUP - 1, (NSUP - 1) % 2, gh[(NSUP - 1) % 2])

    plsc.subcore_barrier()
    pltpu.sync_copy(acc.at[pl.ds(r0, RPT)], accp_hbm.at[cix, pl.ds(r0, RPT)])


_spmm_call = pl.kernel(
    _spmm_body,
    out_type=jax.ShapeDtypeStruct((NC, N, D), f32),
    mesh=_mesh,
    compiler_params=_sc_params,
    scratch_types=[
        pltpu.VMEM((SUP, KS), jnp.int32),
        pltpu.VMEM((SUP, KS), jnp.int32),
        pltpu.VMEM((SUP, KS), jnp.int32),
        pltpu.VMEM((SUP, KS), jnp.int32),
        pltpu.VMEM((SUP, KS, D), f32),
        pltpu.VMEM((SUP, KS, D), f32),
        pltpu.VMEM((ZB, D), f32),
        pltpu.SemaphoreType.DMA,
        pltpu.SemaphoreType.DMA,
        pltpu.SemaphoreType.DMA,
        pltpu.SemaphoreType.DMA,
        pltpu.VMEM_SHARED((N, D), f32),
    ],
)

R = 2048
G = (N + R - 1) // R


def _mm1_body(rep, w1, xw1):
    xw1[...] = jnp.dot(rep[...], w1[...], preferred_element_type=f32)


def _mm1(rep, W1):
    return pl.pallas_call(
        _mm1_body,
        grid=(G,),
        in_specs=[
            pl.BlockSpec((R, D), lambda i: (i, 0)),
            pl.BlockSpec((D, D), lambda i: (0, 0)),
        ],
        out_specs=pl.BlockSpec((R, D), lambda i: (i, 0)),
        out_shape=jax.ShapeDtypeStruct((N, D), f32),
    )(rep, W1)


def _tpart_body(ind, g, w1a_t, w1g_t, b1_t, w1a_c, w1g_c, b1_c, tpt, tpc):
    iv = ind[...]
    gg = g[...][:, None]
    tpt[...] = (jnp.dot(iv, w1a_t[...], preferred_element_type=f32)
                + gg * w1g_t[...] + b1_t[...])
    tpc[...] = (jnp.dot(iv, w1a_c[...], preferred_element_type=f32)
                + gg * w1g_c[...] + b1_c[...])


def _tpart(ind_rep, g, w1a_t, w1g_t, b1_t, w1a_c, w1g_c, b1_c):
    return pl.pallas_call(
        _tpart_body,
        grid=(G,),
        in_specs=[
            pl.BlockSpec((R, D), lambda i: (i, 0)),
            pl.BlockSpec((R,), lambda i: (i,)),
            pl.BlockSpec((D, D), lambda i: (0, 0)),
            pl.BlockSpec((1, D), lambda i: (0, 0)),
            pl.BlockSpec((1, D), lambda i: (0, 0)),
            pl.BlockSpec((D, D), lambda i: (0, 0)),
            pl.BlockSpec((1, D), lambda i: (0, 0)),
            pl.BlockSpec((1, D), lambda i: (0, 0)),
        ],
        out_specs=[
            pl.BlockSpec((R, D), lambda i: (i, 0)),
            pl.BlockSpec((R, D), lambda i: (i, 0)),
        ],
        out_shape=[
            jax.ShapeDtypeStruct((N, D), f32),
            jax.ShapeDtypeStruct((N, D), f32),
        ],
    )(ind_rep, g, w1a_t, w1g_t, b1_t, w1a_c, w1g_c, b1_c)


def _scale1_body(xw1, dega, degb, xs1, dis):
    d = lax.rsqrt(dega[...] + degb[...] + 2.0)
    dis[...] = d
    xs1[...] = d[:, None] * xw1[...]


def _scale1(xw1, dega, degb):
    return pl.pallas_call(
        _scale1_body,
        grid=(G,),
        in_specs=[
            pl.BlockSpec((R, D), lambda i: (i, 0)),
            pl.BlockSpec((R,), lambda i: (i,)),
            pl.BlockSpec((R,), lambda i: (i,)),
        ],
        out_specs=[
            pl.BlockSpec((R, D), lambda i: (i, 0)),
            pl.BlockSpec((R,), lambda i: (i,)),
        ],
        out_shape=[
            jax.ShapeDtypeStruct((N, D), f32),
            jax.ShapeDtypeStruct((N,), f32),
        ],
    )(xw1, dega, degb)


def _prep2_body(acca, accb, xs1, dis, b1, w2, xs2):
    d = dis[...][:, None]
    h1 = jnp.maximum(
        d * (acca[...][0] + accb[...][0] + 2.0 * xs1[...]) + b1[...], 0.0)
    xs2[...] = d * jnp.dot(h1, w2[...], preferred_element_type=f32)


def _prep2(accp, xs1, dis, b1, W2):
    return pl.pallas_call(
        _prep2_body,
        grid=(G,),
        in_specs=[
            pl.BlockSpec((1, R, D), lambda i: (0, i, 0)),
            pl.BlockSpec((1, R, D), lambda i: (1, i, 0)),
            pl.BlockSpec((R, D), lambda i: (i, 0)),
            pl.BlockSpec((R,), lambda i: (i,)),
            pl.BlockSpec((1, D), lambda i: (0, 0)),
            pl.BlockSpec((D, D), lambda i: (0, 0)),
        ],
        out_specs=pl.BlockSpec((R, D), lambda i: (i, 0)),
        out_shape=jax.ShapeDtypeStruct((N, D), f32),
    )(accp, accp, xs1, dis, b1, W2)


def _final_body(tpt, tpc, acca, accb, xs2, dis, b2, tcm,
                wt1b, wt2, bt2, wt3, bt3,
                wc1b, wc2, bc2, wc3, bc3,
                spill, h2o):
    d = dis[...][:, None]
    h2 = jnp.maximum(
        d * (acca[...][0] + accb[...][0] + 2.0 * xs2[...]) + b2[...], 0.0)
    h2o[...] = h2

    def head(tp, w1b, w2_, bb2, w3r, bb3):
        t1 = jnp.maximum(
            tp[...] + jnp.dot(h2, w1b[...], preferred_element_type=f32), 0.0)
        t2 = jnp.maximum(
            jnp.dot(t1, w2_[...], preferred_element_type=f32) + bb2[...], 0.0)
        return jnp.sum(t2 * w3r[...], axis=1) + bb3[...][0, 0]

    st = head(tpt, wt1b, wt2, bt2, wt3, bt3)
    sc = head(tpc, wc1b, wc2, bc2, wc3, bc3)
    spill[...] = jnp.where(tcm[...] == 1, st, sc)


def _final(tpt, tpc, accp, xs2, dis, b2, tcm, wts):
    H2M, H3 = 128, 64
    row = lambda: pl.BlockSpec((R, D), lambda i: (i, 0))
    vec = lambda: pl.BlockSpec((R,), lambda i: (i,))
    full = lambda a, b: pl.BlockSpec((a, b), lambda i: (0, 0))
    return pl.pallas_call(
        _final_body,
        grid=(G,),
        in_specs=[
            row(), row(),
            pl.BlockSpec((1, R, D), lambda i: (0, i, 0)),
            pl.BlockSpec((1, R, D), lambda i: (1, i, 0)),
            row(), vec(), full(1, D), vec(),
            full(D, H2M), full(H2M, H3), full(1, H3), full(1, H3), full(1, 1),
            full(D, H2M), full(H2M, H3), full(1, H3), full(1, H3), full(1, 1),
        ],
        out_specs=[
            pl.BlockSpec((R,), lambda i: (i,)),
            pl.BlockSpec((R, D), lambda i: (i, 0)),
        ],
        out_shape=[
            jax.ShapeDtypeStruct((N,), f32),
            jax.ShapeDtypeStruct((N, D), f32),
        ],
    )(tpt, tpc, accp, accp, xs2, dis, b2, tcm, *wts)


def kernel(ind_rep, rep, tc, g, edge_index, W1, b1, W2, b2,
           Wt1, bt1, Wt2, bt2, Wt3, bt3, Wc1, bc1, Wc2, bc2, Wc3, bc3):
    er = edge_index.astype(jnp.int32).reshape(2, NW * CHS, KS)

    dega, degb = _deg_call(er)
    xw1 = _mm1(rep, W1)
    tpt, tpc = _tpart(ind_rep, g,
                      Wt1[:D], Wt1[2 * D:].reshape(1, D), bt1.reshape(1, D),
                      Wc1[:D], Wc1[2 * D:].reshape(1, D), bc1.reshape(1, D))
    xs1, dis = _scale1(xw1, dega, degb)
    accp1 = _spmm_call(xs1, er)
    xs2 = _prep2(accp1, xs1, dis, b1.reshape(1, D), W2)
    accp2 = _spmm_call(xs2, er)

    wts = (Wt1[D:2 * D], Wt2, bt2.reshape(1, 64),
           Wt3.reshape(1, 64), bt3.reshape(1, 1),
           Wc1[D:2 * D], Wc2, bc2.reshape(1, 64),
           Wc3.reshape(1, 64), bc3.reshape(1, 1))
    spill, h2 = _final(tpt, tpc, accp2, xs2, dis,
                       b2.reshape(1, D), tc.astype(jnp.int32), wts)
    return spill, h2


# speedup vs baseline: 1.2634x; 1.0907x over previous
import jax
import jax.numpy as jnp
from jax import lax
from jax.experimental import pallas as pl
from jax.experimental.pallas import tpu as pltpu
from jax.experimental.pallas import tpu_sc as plsc

N = 10000
E = 320000
D = 128
NC = 2
NS = 16
NW = NC * NS
EPW = E // NW
KS = 80
CHS = EPW // KS
SUP = 2
NSUP = (CHS + SUP - 1) // SUP
RPT = N // NS
Z8 = 624

f32 = jnp.float32

_mesh = plsc.VectorSubcoreMesh(
    core_axis_name="c", subcore_axis_name="s", num_cores=NC, num_subcores=NS)


def _deg_body(er_hbm, degp0_hbm, degp1_hbm, dstall, ones_v, zb, dsem, degacc):
    cix = lax.axis_index("c")
    sid = lax.axis_index("s")
    w = cix * NS + sid
    z0 = sid * Z8
    tail0, tailn = NS * Z8, N - NS * Z8
    for i in range(Z8 // 16):
        zb[pl.ds(i * 16, 16)] = jnp.zeros((16,), f32)
    pltpu.sync_copy(zb, degacc.at[pl.ds(z0, Z8)])

    @pl.when(sid == NS - 1)
    def _():
        pltpu.sync_copy(zb.at[pl.ds(0, tailn)], degacc.at[pl.ds(tail0, tailn)])

    for i in range(KS // 16):
        ones_v[pl.ds(i * 16, 16)] = jnp.ones((16,), f32)
    pltpu.sync_copy(er_hbm.at[1, pl.ds(w * CHS, CHS)], dstall)
    plsc.subcore_barrier()
    WAVE = 8
    for g0 in range(0, CHS, WAVE):
        hs = [pltpu.async_copy(ones_v, degacc.at[dstall.at[ci]], dsem,
                               add=True)
              for ci in range(g0, min(g0 + WAVE, CHS))]
        for h in hs:
            h.wait()

    plsc.subcore_barrier()
    degp = (degp0_hbm, degp1_hbm)
    for c in range(NC):
        @pl.when(cix == c)
        def _():
            pltpu.sync_copy(degacc.at[pl.ds(z0, Z8)],
                            degp[c].at[pl.ds(z0, Z8)])

            @pl.when(sid == NS - 1)
            def _():
                pltpu.sync_copy(degacc.at[pl.ds(tail0, tailn)],
                                degp[c].at[pl.ds(tail0, tailn)])


_sc_params = pltpu.CompilerParams(use_tc_tiling_on_sc=False)

_deg_call = pl.kernel(
    _deg_body,
    out_type=(jax.ShapeDtypeStruct((N,), f32), jax.ShapeDtypeStruct((N,), f32)),
    mesh=_mesh,
    compiler_params=_sc_params,
    scratch_types=[
        pltpu.VMEM((CHS, KS), jnp.int32),
        pltpu.VMEM((KS,), f32),
        pltpu.VMEM((Z8,), f32),
        pltpu.SemaphoreType.DMA,
        pltpu.VMEM_SHARED((N,), f32),
    ],
)


ZB = 25


def _spmm_body(xs_hbm, er_hbm, accp_hbm,
               sidx0, sidx1, didx0, didx1, didx2, didx3, rows0, rows1, zb,
               isem0, isem1, gsem0, gsem1, ssem0, ssem1, acc):
    cix = lax.axis_index("c")
    sid = lax.axis_index("s")
    w = cix * NS + sid
    r0 = sid * RPT
    for j in range(ZB):
        for i in range(D // 16):
            zb[j, pl.ds(i * 16, 16)] = jnp.zeros((16,), f32)
    zhs = [pltpu.async_copy(zb, acc.at[pl.ds(r0 + j * ZB, ZB)], gsem0)
           for j in range(RPT // ZB)]

    sidx = (sidx0, sidx1)
    didx = (didx0, didx1, didx2, didx3)
    rows = (rows0, rows1)
    isems = (isem0, isem1)
    gsems = (gsem0, gsem1)
    ssems = (ssem0, ssem1)
    pend_scat = {0: [], 1: []}

    def nch(si):
        return min(SUP, CHS - si * SUP)

    def fire_idx(si, k):
        row0 = w * CHS + si * SUP
        nc = nch(si)
        return [pltpu.async_copy(er_hbm.at[0, pl.ds(row0, nc)],
                                 sidx[k].at[pl.ds(0, nc)], isems[k]),
                pltpu.async_copy(er_hbm.at[1, pl.ds(row0, nc)],
                                 didx[si % 4].at[pl.ds(0, nc)], isems[k])]

    def fire_gather(si, k, ih):
        for h in pend_scat[k]:
            h.wait()
        pend_scat[k] = []
        for h in ih:
            h.wait()
        return [pltpu.async_copy(xs_hbm.at[sidx[k].at[c]],
                                 rows[k].at[c], gsems[k])
                for c in range(nch(si))]

    def drain(si, k, hs):
        for h in hs:
            h.wait()
        pend_scat[k] = [
            pltpu.async_copy(rows[k].at[c], acc.at[didx[si % 4].at[c]],
                             ssems[k], add=True)
            for c in range(nch(si))]

    ih = fire_idx(0, 0)
    for zh in zhs:
        zh.wait()
    gh = {0: fire_gather(0, 0, ih)}
    ih = fire_idx(1, 1)
    plsc.subcore_barrier()
    for si in range(1, NSUP):
        k = si % 2
        p = 1 - k
        gh[k] = fire_gather(si, k, ih)
        drain(si - 1, p, gh[p])
        if si + 1 < NSUP:
            ih = fire_idx(si + 1, p)
    drain(NSUP - 1, (NSUP - 1) % 2, gh[(NSUP - 1) % 2])
    for k in (0, 1):
        for h in pend_scat[k]:
            h.wait()

    plsc.subcore_barrier()
    pltpu.sync_copy(acc.at[pl.ds(r0, RPT)], accp_hbm.at[cix, pl.ds(r0, RPT)])


_spmm_call = pl.kernel(
    _spmm_body,
    out_type=jax.ShapeDtypeStruct((NC, N, D), f32),
    mesh=_mesh,
    compiler_params=_sc_params,
    scratch_types=[
        pltpu.VMEM((SUP, KS), jnp.int32),
        pltpu.VMEM((SUP, KS), jnp.int32),
        pltpu.VMEM((SUP, KS), jnp.int32),
        pltpu.VMEM((SUP, KS), jnp.int32),
        pltpu.VMEM((SUP, KS), jnp.int32),
        pltpu.VMEM((SUP, KS), jnp.int32),
        pltpu.VMEM((SUP, KS, D), f32),
        pltpu.VMEM((SUP, KS, D), f32),
        pltpu.VMEM((ZB, D), f32),
        pltpu.SemaphoreType.DMA,
        pltpu.SemaphoreType.DMA,
        pltpu.SemaphoreType.DMA,
        pltpu.SemaphoreType.DMA,
        pltpu.SemaphoreType.DMA,
        pltpu.SemaphoreType.DMA,
        pltpu.VMEM_SHARED((N, D), f32),
    ],
)

R = 2048
G = (N + R - 1) // R


def _mm1_body(rep, w1, xw1):
    xw1[...] = jnp.dot(rep[...], w1[...], preferred_element_type=f32)


def _mm1(rep, W1):
    return pl.pallas_call(
        _mm1_body,
        grid=(G,),
        in_specs=[
            pl.BlockSpec((R, D), lambda i: (i, 0)),
            pl.BlockSpec((D, D), lambda i: (0, 0)),
        ],
        out_specs=pl.BlockSpec((R, D), lambda i: (i, 0)),
        out_shape=jax.ShapeDtypeStruct((N, D), f32),
    )(rep, W1)


def _tpart_body(ind, g, w1a_t, w1g_t, b1_t, w1a_c, w1g_c, b1_c, tpt, tpc):
    iv = ind[...]
    gg = g[...][:, None]
    tpt[...] = (jnp.dot(iv, w1a_t[...], preferred_element_type=f32)
                + gg * w1g_t[...] + b1_t[...])
    tpc[...] = (jnp.dot(iv, w1a_c[...], preferred_element_type=f32)
                + gg * w1g_c[...] + b1_c[...])


def _tpart(ind_rep, g, w1a_t, w1g_t, b1_t, w1a_c, w1g_c, b1_c):
    return pl.pallas_call(
        _tpart_body,
        grid=(G,),
        in_specs=[
            pl.BlockSpec((R, D), lambda i: (i, 0)),
            pl.BlockSpec((R,), lambda i: (i,)),
            pl.BlockSpec((D, D), lambda i: (0, 0)),
            pl.BlockSpec((1, D), lambda i: (0, 0)),
            pl.BlockSpec((1, D), lambda i: (0, 0)),
            pl.BlockSpec((D, D), lambda i: (0, 0)),
            pl.BlockSpec((1, D), lambda i: (0, 0)),
            pl.BlockSpec((1, D), lambda i: (0, 0)),
        ],
        out_specs=[
            pl.BlockSpec((R, D), lambda i: (i, 0)),
            pl.BlockSpec((R, D), lambda i: (i, 0)),
        ],
        out_shape=[
            jax.ShapeDtypeStruct((N, D), f32),
            jax.ShapeDtypeStruct((N, D), f32),
        ],
    )(ind_rep, g, w1a_t, w1g_t, b1_t, w1a_c, w1g_c, b1_c)


def _scale1_body(xw1, dega, degb, xs1, dis):
    d = lax.rsqrt(dega[...] + degb[...] + 2.0)
    dis[...] = d
    xs1[...] = d[:, None] * xw1[...]


def _scale1(xw1, dega, degb):
    return pl.pallas_call(
        _scale1_body,
        grid=(G,),
        in_specs=[
            pl.BlockSpec((R, D), lambda i: (i, 0)),
            pl.BlockSpec((R,), lambda i: (i,)),
            pl.BlockSpec((R,), lambda i: (i,)),
        ],
        out_specs=[
            pl.BlockSpec((R, D), lambda i: (i, 0)),
            pl.BlockSpec((R,), lambda i: (i,)),
        ],
        out_shape=[
            jax.ShapeDtypeStruct((N, D), f32),
            jax.ShapeDtypeStruct((N,), f32),
        ],
    )(xw1, dega, degb)


def _prep2_body(acca, accb, xs1, dis, b1, w2, xs2):
    d = dis[...][:, None]
    h1 = jnp.maximum(
        d * (acca[...][0] + accb[...][0] + 2.0 * xs1[...]) + b1[...], 0.0)
    xs2[...] = d * jnp.dot(h1, w2[...], preferred_element_type=f32)


def _prep2(accp, xs1, dis, b1, W2):
    return pl.pallas_call(
        _prep2_body,
        grid=(G,),
        in_specs=[
            pl.BlockSpec((1, R, D), lambda i: (0, i, 0)),
            pl.BlockSpec((1, R, D), lambda i: (1, i, 0)),
            pl.BlockSpec((R, D), lambda i: (i, 0)),
            pl.BlockSpec((R,), lambda i: (i,)),
            pl.BlockSpec((1, D), lambda i: (0, 0)),
            pl.BlockSpec((D, D), lambda i: (0, 0)),
        ],
        out_specs=pl.BlockSpec((R, D), lambda i: (i, 0)),
        out_shape=jax.ShapeDtypeStruct((N, D), f32),
    )(accp, accp, xs1, dis, b1, W2)


def _final_body(tpt, tpc, acca, accb, xs2, dis, b2, tcm,
                wt1b, wt2, bt2, wt3, bt3,
                wc1b, wc2, bc2, wc3, bc3,
                spill, h2o):
    d = dis[...][:, None]
    h2 = jnp.maximum(
        d * (acca[...][0] + accb[...][0] + 2.0 * xs2[...]) + b2[...], 0.0)
    h2o[...] = h2

    def head(tp, w1b, w2_, bb2, w3r, bb3):
        t1 = jnp.maximum(
            tp[...] + jnp.dot(h2, w1b[...], preferred_element_type=f32), 0.0)
        t2 = jnp.maximum(
            jnp.dot(t1, w2_[...], preferred_element_type=f32) + bb2[...], 0.0)
        return jnp.sum(t2 * w3r[...], axis=1) + bb3[...][0, 0]

    st = head(tpt, wt1b, wt2, bt2, wt3, bt3)
    sc = head(tpc, wc1b, wc2, bc2, wc3, bc3)
    spill[...] = jnp.where(tcm[...] == 1, st, sc)


def _final(tpt, tpc, accp, xs2, dis, b2, tcm, wts):
    H2M, H3 = 128, 64
    row = lambda: pl.BlockSpec((R, D), lambda i: (i, 0))
    vec = lambda: pl.BlockSpec((R,), lambda i: (i,))
    full = lambda a, b: pl.BlockSpec((a, b), lambda i: (0, 0))
    return pl.pallas_call(
        _final_body,
        grid=(G,),
        in_specs=[
            row(), row(),
            pl.BlockSpec((1, R, D), lambda i: (0, i, 0)),
            pl.BlockSpec((1, R, D), lambda i: (1, i, 0)),
            row(), vec(), full(1, D), vec(),
            full(D, H2M), full(H2M, H3), full(1, H3), full(1, H3), full(1, 1),
            full(D, H2M), full(H2M, H3), full(1, H3), full(1, H3), full(1, 1),
        ],
        out_specs=[
            pl.BlockSpec((R,), lambda i: (i,)),
            pl.BlockSpec((R, D), lambda i: (i, 0)),
        ],
        out_shape=[
            jax.ShapeDtypeStruct((N,), f32),
            jax.ShapeDtypeStruct((N, D), f32),
        ],
    )(tpt, tpc, accp, accp, xs2, dis, b2, tcm, *wts)


def kernel(ind_rep, rep, tc, g, edge_index, W1, b1, W2, b2,
           Wt1, bt1, Wt2, bt2, Wt3, bt3, Wc1, bc1, Wc2, bc2, Wc3, bc3):
    er = edge_index.astype(jnp.int32).reshape(2, NW * CHS, KS)

    dega, degb = _deg_call(er)
    xw1 = _mm1(rep, W1)
    tpt, tpc = _tpart(ind_rep, g,
                      Wt1[:D], Wt1[2 * D:].reshape(1, D), bt1.reshape(1, D),
                      Wc1[:D], Wc1[2 * D:].reshape(1, D), bc1.reshape(1, D))
    xs1, dis = _scale1(xw1, dega, degb)
    accp1 = _spmm_call(xs1, er)
    xs2 = _prep2(accp1, xs1, dis, b1.reshape(1, D), W2)
    accp2 = _spmm_call(xs2, er)

    wts = (Wt1[D:2 * D], Wt2, bt2.reshape(1, 64),
           Wt3.reshape(1, 64), bt3.reshape(1, 1),
           Wc1[D:2 * D], Wc2, bc2.reshape(1, 64),
           Wc3.reshape(1, 64), bc3.reshape(1, 1))
    spill, h2 = _final(tpt, tpc, accp2, xs2, dis,
                       b2.reshape(1, D), tc.astype(jnp.int32), wts)
    return spill, h2
